# baseline (speedup 1.0000x reference)
import functools

import jax
import jax.numpy as jnp
from jax import lax
from jax.experimental import pallas as pl
from jax.experimental.pallas import tpu as pltpu
from jax.experimental.pallas import tpu_sc as plsc

MARGIN = 0.2
N = 4096
TC_ROWS = 2560
TC_BLK = 256
NC, NS, L = 2, 16, 16
NW = NC * NS
SC_ROWS = N - TC_ROWS
ROWS_PER_W = SC_ROWS // NW
R = 8
NBUF = 3
NCHUNK = ROWS_PER_W // R
SLICES = 8
COLS_PER_IT = SLICES * L
NEG_INF = float("-inf")

_mesh = plsc.VectorSubcoreMesh(core_axis_name="c", subcore_axis_name="s")


@functools.partial(
    pl.kernel,
    out_type=jax.ShapeDtypeStruct((NW, L), jnp.float32),
    mesh=_mesh,
    compiler_params=pltpu.CompilerParams(needs_layout_passes=False),
    scratch_types=[
        pltpu.VMEM((NBUF * R, N), jnp.float32),
        pltpu.VMEM((L,), jnp.float32),
        [pltpu.SemaphoreType.DMA for _ in range(NBUF)],
    ],
)
def _sc_partials(mat_hbm, out_hbm, buf, obuf, sems):
    wid = lax.axis_index("s") * NC + lax.axis_index("c")
    base = TC_ROWS + wid * ROWS_PER_W

    for b in range(NBUF):
        pltpu.async_copy(mat_hbm.at[pl.ds(base + b * R, R)],
                         buf.at[pl.ds(b * R, R)], sems[b])

    lanes = lax.iota(jnp.int32, L)
    neg_fill = jnp.full((L,), NEG_INF, jnp.float32)

    def body(c, loss):
        bsel = lax.rem(c, NBUF)
        boff = bsel * R
        row0 = base + c * R
        for b in range(NBUF):
            @pl.when(bsel == b)
            def _():
                pltpu.make_async_copy(
                    mat_hbm.at[pl.ds(row0, R)],
                    buf.at[pl.ds(b * R, R)], sems[b]).wait()

        poss = []
        for r in range(R):
            col = row0 + r
            sb = (col // L) * L
            tgt = col - sb
            v = buf[boff + r, pl.ds(sb, L)]
            m = lanes == tgt
            poss.append(jnp.max(jnp.where(m, v, neg_fill)))
            buf[boff + r, pl.ds(sb, L)] = jnp.where(m, neg_fill, v)

        def inner(k, accs):
            c0 = k * COLS_PER_IT
            out = []
            for r in range(R):
                vs = [buf[boff + r, pl.ds(c0 + s * L, L)]
                      for s in range(SLICES)]
                while len(vs) > 1:
                    vs = [jnp.maximum(vs[i], vs[i + 1])
                          for i in range(0, len(vs), 2)]
                out.append(jnp.maximum(accs[r], vs[0]))
            return tuple(out)

        accs = lax.fori_loop(0, N // COLS_PER_IT, inner, (neg_fill,) * R)
        for r in range(R):
            loss = loss + jnp.maximum(jnp.max(accs[r]) - poss[r] + MARGIN,
                                      0.0)

        for b in range(NBUF):
            @pl.when((bsel == b) & (c + NBUF < NCHUNK))
            def _():
                pltpu.async_copy(
                    mat_hbm.at[pl.ds(row0 + NBUF * R, R)],
                    buf.at[pl.ds(b * R, R)], sems[b])
        return loss

    loss = lax.fori_loop(0, NCHUNK, body, jnp.float32(0.0))
    obuf[...] = jnp.broadcast_to(loss, (L,))
    pltpu.sync_copy(obuf, out_hbm.at[wid])


def _tc_body(mat_ref, out_ref):
    i = pl.program_id(0)
    rl = lax.broadcasted_iota(jnp.int32, (TC_BLK, TC_BLK), 0)
    cl = lax.broadcasted_iota(jnp.int32, (TC_BLK, TC_BLK), 1)
    dmask = rl == cl
    sub = mat_ref[:, pl.ds(i * TC_BLK, TC_BLK)]
    pos = jnp.max(jnp.where(dmask, sub, NEG_INF), axis=1)
    mat_ref[:, pl.ds(i * TC_BLK, TC_BLK)] = jnp.where(dmask, NEG_INF, sub)
    neg = jnp.max(mat_ref[...], axis=1)
    part = jnp.sum(jnp.maximum(neg - pos + MARGIN, 0.0))

    @pl.when(i == 0)
    def _():
        out_ref[0, 0] = 0.0

    out_ref[0, 0] += part


_tc_loss = pl.pallas_call(
    _tc_body,
    grid=(TC_ROWS // TC_BLK,),
    in_specs=[pl.BlockSpec((TC_BLK, N), lambda i: (i, 0))],
    out_specs=pl.BlockSpec(memory_space=pltpu.SMEM),
    out_shape=jax.ShapeDtypeStruct((1, 1), jnp.float32),
)


def kernel(matrix):
    tc = _tc_loss(matrix)
    sc = _sc_partials(matrix)
    return tc[0, 0] + jnp.sum(sc[:, 0])


# speedup vs baseline: 1.0356x; 1.0356x over previous
import functools

import jax
import jax.numpy as jnp
from jax import lax
from jax.experimental import pallas as pl
from jax.experimental.pallas import tpu as pltpu
from jax.experimental.pallas import tpu_sc as plsc

MARGIN = 0.2
N = 4096
TC_ROWS = 3072
TC_BLK = 256
NC, NS, L = 2, 16, 16
NW = NC * NS
SC_ROWS = N - TC_ROWS
ROWS_PER_W = SC_ROWS // NW
R = 8
NBUF = 3
NCHUNK = ROWS_PER_W // R
SLICES = 8
COLS_PER_IT = SLICES * L
NEG_INF = float("-inf")

_mesh = plsc.VectorSubcoreMesh(core_axis_name="c", subcore_axis_name="s")


@functools.partial(
    pl.kernel,
    out_type=jax.ShapeDtypeStruct((NW, L), jnp.float32),
    mesh=_mesh,
    compiler_params=pltpu.CompilerParams(needs_layout_passes=False),
    scratch_types=[
        pltpu.VMEM((NBUF * R, N), jnp.float32),
        pltpu.VMEM((L,), jnp.float32),
        [pltpu.SemaphoreType.DMA for _ in range(NBUF)],
    ],
)
def _sc_partials(mat_hbm, out_hbm, buf, obuf, sems):
    wid = lax.axis_index("s") * NC + lax.axis_index("c")
    base = TC_ROWS + wid * ROWS_PER_W

    for b in range(NBUF):
        pltpu.async_copy(mat_hbm.at[pl.ds(base + b * R, R)],
                         buf.at[pl.ds(b * R, R)], sems[b])

    lanes = lax.iota(jnp.int32, L)
    neg_fill = jnp.full((L,), NEG_INF, jnp.float32)

    def body(c, loss):
        bsel = lax.rem(c, NBUF)
        boff = bsel * R
        row0 = base + c * R
        for b in range(NBUF):
            @pl.when(bsel == b)
            def _():
                pltpu.make_async_copy(
                    mat_hbm.at[pl.ds(row0, R)],
                    buf.at[pl.ds(b * R, R)], sems[b]).wait()

        poss = []
        for r in range(R):
            col = row0 + r
            sb = (col // L) * L
            tgt = col - sb
            v = buf[boff + r, pl.ds(sb, L)]
            m = lanes == tgt
            poss.append(jnp.max(jnp.where(m, v, neg_fill)))
            buf[boff + r, pl.ds(sb, L)] = jnp.where(m, neg_fill, v)

        def inner(k, accs):
            c0 = k * COLS_PER_IT
            out = []
            for r in range(R):
                vs = [buf[boff + r, pl.ds(c0 + s * L, L)]
                      for s in range(SLICES)]
                while len(vs) > 1:
                    vs = [jnp.maximum(vs[i], vs[i + 1])
                          for i in range(0, len(vs), 2)]
                out.append(jnp.maximum(accs[r], vs[0]))
            return tuple(out)

        accs = lax.fori_loop(0, N // COLS_PER_IT, inner, (neg_fill,) * R)
        for r in range(R):
            loss = loss + jnp.maximum(jnp.max(accs[r]) - poss[r] + MARGIN,
                                      0.0)

        for b in range(NBUF):
            @pl.when((bsel == b) & (c + NBUF < NCHUNK))
            def _():
                pltpu.async_copy(
                    mat_hbm.at[pl.ds(row0 + NBUF * R, R)],
                    buf.at[pl.ds(b * R, R)], sems[b])
        return loss

    loss = lax.fori_loop(0, NCHUNK, body, jnp.float32(0.0))
    obuf[...] = jnp.broadcast_to(loss, (L,))
    pltpu.sync_copy(obuf, out_hbm.at[wid])


def _tc_body(mat_ref, out_ref):
    i = pl.program_id(0)
    rl = lax.broadcasted_iota(jnp.int32, (TC_BLK, TC_BLK), 0)
    cl = lax.broadcasted_iota(jnp.int32, (TC_BLK, TC_BLK), 1)
    dmask = rl == cl
    sub = mat_ref[:, pl.ds(i * TC_BLK, TC_BLK)]
    pos = jnp.max(jnp.where(dmask, sub, NEG_INF), axis=1)
    mat_ref[:, pl.ds(i * TC_BLK, TC_BLK)] = jnp.where(dmask, NEG_INF, sub)
    neg = jnp.max(mat_ref[...], axis=1)
    part = jnp.sum(jnp.maximum(neg - pos + MARGIN, 0.0))

    @pl.when(i == 0)
    def _():
        out_ref[0, 0] = 0.0

    out_ref[0, 0] += part


_tc_loss = pl.pallas_call(
    _tc_body,
    grid=(TC_ROWS // TC_BLK,),
    in_specs=[pl.BlockSpec((TC_BLK, N), lambda i: (i, 0))],
    out_specs=pl.BlockSpec(memory_space=pltpu.SMEM),
    out_shape=jax.ShapeDtypeStruct((1, 1), jnp.float32),
)


def kernel(matrix):
    tc = _tc_loss(matrix)
    sc = _sc_partials(matrix)
    return tc[0, 0] + jnp.sum(sc[:, 0])
